# Initial kernel scaffold; baseline (speedup 1.0000x reference)
import jax
import jax.numpy as jnp
from jax.experimental import pallas as pl


def kernel(feats, pe_x_table, pe_y_table, glb_table):
    raise NotImplementedError("write your pallas kernel here")


# speedup vs baseline: 1.1456x; 1.1456x over previous
import jax
import jax.numpy as jnp
from jax.experimental import pallas as pl


def _pe_kernel(feats_ref, pe_x_ref, pe_y_ref, glb_ref, out_ref):
    c = feats_ref.shape[1]
    hw = feats_ref.shape[2]
    h = pe_y_ref.shape[0]
    w = pe_x_ref.shape[0]
    dim = pe_x_ref.shape[1]

    x = feats_ref[0]
    xt = jnp.transpose(x, (1, 0))

    pe_x = pe_x_ref[...]
    pe_y = pe_y_ref[...]
    pe_x_full = jnp.broadcast_to(pe_x[None, :, :], (h, w, dim)).reshape(hw, dim)
    pe_y_full = jnp.broadcast_to(pe_y[:, None, :], (h, w, dim)).reshape(hw, dim)
    pe = jnp.concatenate([pe_x_full, pe_y_full], axis=1)

    out_ref[0, 1:, :] = xt + pe
    out_ref[0, 0:1, :] = glb_ref[...]


def kernel(feats, pe_x_table, pe_y_table, glb_table):
    b, c, h, w = feats.shape
    hw = h * w
    feats2 = feats.reshape(b, c, hw)

    out = pl.pallas_call(
        _pe_kernel,
        grid=(b,),
        in_specs=[
            pl.BlockSpec((1, c, hw), lambda i: (i, 0, 0)),
            pl.BlockSpec((w, pe_x_table.shape[1]), lambda i: (0, 0)),
            pl.BlockSpec((h, pe_y_table.shape[1]), lambda i: (0, 0)),
            pl.BlockSpec((1, c), lambda i: (0, 0)),
        ],
        out_specs=pl.BlockSpec((1, 1 + hw, c), lambda i: (i, 0, 0)),
        out_shape=jax.ShapeDtypeStruct((b, 1 + hw, c), feats.dtype),
    )(feats2, pe_x_table, pe_y_table, glb_table)
    return out
